# Initial kernel scaffold; baseline (speedup 1.0000x reference)
import jax
import jax.numpy as jnp
from jax.experimental import pallas as pl


def kernel(user_emb, item_emb, edge_index, edge_values):
    raise NotImplementedError("write your pallas kernel here")


# speedup vs baseline: 4.6643x; 4.6643x over previous
import functools

import jax
import jax.numpy as jnp
from jax import lax
from jax.experimental import pallas as pl
from jax.experimental.pallas import tpu as pltpu
from jax.experimental.pallas import tpu_sc as plsc

N_U = 50000
N_I = 50000
N = N_U + N_I
N_PAD = 100096
D = 32
H = 16
E_RAW = 1600000
CH = 128
CPB = 4
BLK = CH * CPB
N_TILES = 16
CPT = 800
NB = CPT // CPB
E_PAD = N_TILES * CPT * CH
RPT = N_PAD // N_TILES
EZ = 184
NEZ = RPT // EZ


def _sc_propagate(emb2, colX, rowX, valE, zeros):
    mesh = plsc.VectorSubcoreMesh(core_axis_name="c", subcore_axis_name="s")
    f32 = jnp.float32
    out_types = (
        jax.ShapeDtypeStruct((2 * N_PAD, H), f32),
        jax.ShapeDtypeStruct((2 * N_PAD, H), f32),
        jax.ShapeDtypeStruct((2 * N_PAD, H), f32),
    )
    scratch = [
        pltpu.VMEM_SHARED((N_PAD, H), f32),
        pltpu.VMEM((CPB, CH), jnp.int32),
        pltpu.VMEM((CPB, CH), jnp.int32),
        pltpu.VMEM((BLK, H), f32),
        pltpu.VMEM((BLK, H), f32),
        pltpu.VMEM((EZ, H), f32),
        pltpu.VMEM((EZ, H), f32),
        pltpu.VMEM((EZ, H), f32),
        pltpu.VMEM((EZ, H), f32),
        pltpu.SemaphoreType.DMA,
    ]

    @functools.partial(
        pl.kernel, out_type=out_types, mesh=mesh, scratch_types=scratch,
        compiler_params=pltpu.CompilerParams(use_tc_tiling_on_sc=False))
    def body(emb2_h, colX_h, rowX_h, valE_h, zeros_h, sum_h, l1_h, l2_h,
             acc, colb, rowb, rowsb, valrows, vA, vB, vC, vD, sem):
        c = lax.axis_index("c")
        s = lax.axis_index("s")
        tile_lo = s * RPT

        def do_layer(src_h, dst_h, is_last):
            pltpu.sync_copy(zeros_h, acc.at[pl.ds(tile_lo, RPT)])
            plsc.subcore_barrier()

            def block_body(b, carry):
                cb = s * CPT + b * CPB
                ccb = c * (E_PAD // CH) + cb
                eb = cb * CH
                pltpu.sync_copy(colX_h.at[pl.ds(ccb, CPB)], colb)
                pltpu.sync_copy(rowX_h.at[pl.ds(cb, CPB)], rowb)
                pltpu.sync_copy(valE_h.at[pl.ds(eb, BLK)], valrows)
                cps = [
                    pltpu.async_copy(src_h.at[colb.at[j]],
                                     rowsb.at[pl.ds(j * CH, CH)], sem)
                    for j in range(CPB)
                ]
                for cp in cps:
                    cp.wait()

                def mul_body(i, carry2):
                    rowsb[i, :] = rowsb[i, :] * valrows[i, :]
                    return carry2
                lax.fori_loop(0, BLK, mul_body, 0)

                for j in range(CPB):
                    pltpu.sync_copy(rowsb.at[pl.ds(j * CH, CH)],
                                    acc.at[rowb.at[j]], add=True)
                return carry

            lax.fori_loop(0, NB, block_body, 0)
            plsc.subcore_barrier()

            if not is_last:
                pltpu.sync_copy(acc.at[pl.ds(tile_lo, RPT)],
                                dst_h.at[pl.ds(c * N_PAD + tile_lo, RPT)])
            else:
                def ep_body(z, carry):
                    bl = tile_lo + z * EZ
                    bg = c * N_PAD + bl
                    pltpu.sync_copy(acc.at[pl.ds(bl, EZ)], vA)
                    pltpu.sync_copy(emb2_h.at[pl.ds(bg, EZ)], vB)
                    pltpu.sync_copy(l1_h.at[pl.ds(bg, EZ)], vC)
                    pltpu.sync_copy(l2_h.at[pl.ds(bg, EZ)], vD)

                    def cmb(i, carry2):
                        vA[i, :] = (vA[i, :] + vB[i, :] + vC[i, :]
                                    + vD[i, :]) * 0.25
                        return carry2
                    lax.fori_loop(0, EZ, cmb, 0)
                    pltpu.sync_copy(vA, sum_h.at[pl.ds(bg, EZ)])
                    return carry
                lax.fori_loop(0, NEZ, ep_body, 0)
            plsc.subcore_barrier()

        do_layer(emb2_h, l1_h, False)
        do_layer(l1_h, l2_h, False)
        do_layer(l2_h, None, True)

    return body(emb2, colX, rowX, valE, zeros)


def kernel(user_emb, item_emb, edge_index, edge_values):
    emb = jnp.concatenate([user_emb, item_emb], axis=0)
    padrows = jnp.zeros((N_PAD - N, H), jnp.float32)
    emb2 = jnp.concatenate(
        [emb[:, :H], padrows, emb[:, H:], padrows], axis=0)
    row = edge_index[0]
    col = edge_index[1]
    pad = E_PAD - E_RAW
    row_p = jnp.concatenate([row, jnp.zeros((pad,), jnp.int32)])
    col_p = jnp.concatenate([col, jnp.zeros((pad,), jnp.int32)])
    val_p = jnp.concatenate([edge_values, jnp.zeros((pad,), jnp.float32)])
    colX = jnp.concatenate([col_p, col_p + N_PAD]).reshape(2 * E_PAD // CH, CH)
    rowX = row_p.reshape(E_PAD // CH, CH)
    valE = jnp.broadcast_to(val_p[:, None], (E_PAD, H))
    zeros = jnp.zeros((RPT, H), jnp.float32)
    sum2, _l1, _l2 = _sc_propagate(emb2, colX, rowX, valE, zeros)
    final = jnp.stack([sum2[:N], sum2[N_PAD:N_PAD + N]], axis=1).reshape(N, D)
    return final[:N_U], final[N_U:]


# speedup vs baseline: 7.5718x; 1.6233x over previous
import functools

import jax
import jax.numpy as jnp
from jax import lax
from jax.experimental import pallas as pl
from jax.experimental.pallas import tpu as pltpu
from jax.experimental.pallas import tpu_sc as plsc

_BCAST_DNUMS = lax.GatherDimensionNumbers(
    offset_dims=(), collapsed_slice_dims=(0,), start_index_map=(0,))


def _bcast_lane(v16, lane):
    idx = jnp.full((16, 1), lane, dtype=jnp.int32)
    return lax.gather(v16, idx, _BCAST_DNUMS, (1,),
                      mode=lax.GatherScatterMode.PROMISE_IN_BOUNDS)


N_U = 50000
N_I = 50000
N = N_U + N_I
N_PAD = 100096
D = 32
H = 16
E_RAW = 1600000
CH = 128
CPB = 4
BLK = CH * CPB
N_TILES = 16
CPT = 800
NB = CPT // CPB
E_PAD = N_TILES * CPT * CH
RPT = N_PAD // N_TILES
EZ = 184
NEZ = RPT // EZ


def _sc_propagate(emb2, colX, rowX, valE, zeros):
    mesh = plsc.VectorSubcoreMesh(core_axis_name="c", subcore_axis_name="s")
    f32 = jnp.float32
    out_types = (
        jax.ShapeDtypeStruct((2 * N_PAD, H), f32),
        jax.ShapeDtypeStruct((2 * N_PAD, H), f32),
        jax.ShapeDtypeStruct((2 * N_PAD, H), f32),
    )
    scratch = [
        pltpu.VMEM_SHARED((N_PAD, H), f32),
        pltpu.VMEM((CPB, CH), jnp.int32),
        pltpu.VMEM((CPB, CH), jnp.int32),
        pltpu.VMEM((BLK, H), f32),
        pltpu.VMEM((BLK // 16, 16), f32),
        pltpu.VMEM((EZ, H), f32),
        pltpu.VMEM((EZ, H), f32),
        pltpu.VMEM((EZ, H), f32),
        pltpu.VMEM((EZ, H), f32),
        pltpu.SemaphoreType.DMA,
    ]

    @functools.partial(
        pl.kernel, out_type=out_types, mesh=mesh, scratch_types=scratch,
        compiler_params=pltpu.CompilerParams(use_tc_tiling_on_sc=False))
    def body(emb2_h, colX_h, rowX_h, valX_h, zeros_h, sum_h, l1_h, l2_h,
             acc, colb, rowb, rowsb, valb, vA, vB, vC, vD, sem):
        c = lax.axis_index("c")
        s = lax.axis_index("s")
        tile_lo = s * RPT

        def do_layer(src_h, dst_h, is_last):
            pltpu.sync_copy(zeros_h, acc.at[pl.ds(tile_lo, RPT)])
            plsc.subcore_barrier()

            def block_body(b, carry):
                cb = s * CPT + b * CPB
                ccb = c * (E_PAD // CH) + cb
                eb = cb * CH
                pltpu.sync_copy(colX_h.at[pl.ds(ccb, CPB)], colb)
                pltpu.sync_copy(rowX_h.at[pl.ds(cb, CPB)], rowb)
                pltpu.sync_copy(valX_h.at[pl.ds(cb * (CH // 16), BLK // 16)],
                                valb)
                cps = [
                    pltpu.async_copy(src_h.at[colb.at[j]],
                                     rowsb.at[pl.ds(j * CH, CH)], sem)
                    for j in range(CPB)
                ]
                for cp in cps:
                    cp.wait()

                def mul_body(g, carry2):
                    v16 = valb[g, :]
                    for l in range(16):
                        b16 = _bcast_lane(v16, l)
                        e = g * 16 + l
                        rowsb[e, :] = rowsb[e, :] * b16
                    return carry2
                lax.fori_loop(0, BLK // 16, mul_body, 0)

                for j in range(CPB):
                    pltpu.sync_copy(rowsb.at[pl.ds(j * CH, CH)],
                                    acc.at[rowb.at[j]], add=True)
                return carry

            lax.fori_loop(0, NB, block_body, 0)
            plsc.subcore_barrier()

            if not is_last:
                pltpu.sync_copy(acc.at[pl.ds(tile_lo, RPT)],
                                dst_h.at[pl.ds(c * N_PAD + tile_lo, RPT)])
            else:
                def ep_body(z, carry):
                    bl = tile_lo + z * EZ
                    bg = c * N_PAD + bl
                    pltpu.sync_copy(acc.at[pl.ds(bl, EZ)], vA)
                    pltpu.sync_copy(emb2_h.at[pl.ds(bg, EZ)], vB)
                    pltpu.sync_copy(l1_h.at[pl.ds(bg, EZ)], vC)
                    pltpu.sync_copy(l2_h.at[pl.ds(bg, EZ)], vD)

                    def cmb(i, carry2):
                        vA[i, :] = (vA[i, :] + vB[i, :] + vC[i, :]
                                    + vD[i, :]) * 0.25
                        return carry2
                    lax.fori_loop(0, EZ, cmb, 0)
                    pltpu.sync_copy(vA, sum_h.at[pl.ds(bg, EZ)])
                    return carry
                lax.fori_loop(0, NEZ, ep_body, 0)
            plsc.subcore_barrier()

        do_layer(emb2_h, l1_h, False)
        do_layer(l1_h, l2_h, False)
        do_layer(l2_h, None, True)

    return body(emb2, colX, rowX, valE, zeros)


def kernel(user_emb, item_emb, edge_index, edge_values):
    emb = jnp.concatenate([user_emb, item_emb], axis=0)
    padrows = jnp.zeros((N_PAD - N, H), jnp.float32)
    emb2 = jnp.concatenate(
        [emb[:, :H], padrows, emb[:, H:], padrows], axis=0)
    row = edge_index[0]
    col = edge_index[1]
    pad = E_PAD - E_RAW
    row_p = jnp.concatenate([row, jnp.zeros((pad,), jnp.int32)])
    col_p = jnp.concatenate([col, jnp.zeros((pad,), jnp.int32)])
    val_p = jnp.concatenate([edge_values, jnp.zeros((pad,), jnp.float32)])
    colX = jnp.concatenate([col_p, col_p + N_PAD]).reshape(2 * E_PAD // CH, CH)
    rowX = row_p.reshape(E_PAD // CH, CH)
    valX = val_p.reshape(E_PAD // 16, 16)
    zeros = jnp.zeros((RPT, H), jnp.float32)
    sum2, _l1, _l2 = _sc_propagate(emb2, colX, rowX, valX, zeros)
    final = jnp.stack([sum2[:N], sum2[N_PAD:N_PAD + N]], axis=1).reshape(N, D)
    return final[:N_U], final[N_U:]


# speedup vs baseline: 13.6655x; 1.8048x over previous
import functools

import jax
import jax.numpy as jnp
from jax import lax
from jax.experimental import pallas as pl
from jax.experimental.pallas import tpu as pltpu
from jax.experimental.pallas import tpu_sc as plsc

_BCAST_DNUMS = lax.GatherDimensionNumbers(
    offset_dims=(), collapsed_slice_dims=(0,), start_index_map=(0,))


def _bcast_lane(v16, lane):
    idx = jnp.full((16, 1), lane, dtype=jnp.int32)
    return lax.gather(v16, idx, _BCAST_DNUMS, (1,),
                      mode=lax.GatherScatterMode.PROMISE_IN_BOUNDS)


N_U = 50000
N_I = 50000
N = N_U + N_I
N_PAD = 100096
D = 32
H = 16
E_RAW = 1600000
CH = 128
CPB = 2
BLK = CH * CPB
N_TILES = 16
CPT = 800
NB = CPT // CPB
E_PAD = N_TILES * CPT * CH
RPT = N_PAD // N_TILES
EZ = 136
NEZ = RPT // EZ
GB = CPB * CH * H * 4
LB = 3 * CPB * CH * 4


def _sc_propagate(emb2, colX, rowX, valX, zeros):
    mesh = plsc.VectorSubcoreMesh(core_axis_name="c", subcore_axis_name="s")
    f32 = jnp.float32
    out_types = (
        jax.ShapeDtypeStruct((2 * N_PAD, H), f32),
        jax.ShapeDtypeStruct((2 * N_PAD, H), f32),
        jax.ShapeDtypeStruct((2 * N_PAD, H), f32),
    )
    scratch = (
        [pltpu.VMEM_SHARED((N_PAD, H), f32)]
        + [pltpu.VMEM((4, CPB, CH), jnp.int32)]
        + [pltpu.VMEM((8, CPB, CH), jnp.int32)]
        + [pltpu.VMEM((4, BLK // 16, 16), f32)]
        + [pltpu.VMEM((4, BLK, H), f32)]
        + [pltpu.VMEM((EZ, H), f32)] * 2
        + [pltpu.SemaphoreType.DMA] * 12
    )

    @functools.partial(
        pl.kernel, out_type=out_types, mesh=mesh, scratch_types=scratch,
        compiler_params=pltpu.CompilerParams(use_tc_tiling_on_sc=False))
    def body(emb2_h, colX_h, rowX_h, valX_h, zeros_h, sum_h, l1_h, l2_h,
             acc, colb, rowb, valb, rows, vA, vB,
             sg0, sg1, sg2, sg3, ss0, ss1, ss2, ss3, sl0, sl1, sl2, sl3):
        semG = [sg0, sg1, sg2, sg3]
        semS = [ss0, ss1, ss2, ss3]
        semL = [sl0, sl1, sl2, sl3]
        c = lax.axis_index("c")
        s = lax.axis_index("s")
        tile_lo = s * RPT
        base_cb = s * CPT

        def fire_loads(b, p4, p8):
            cb = base_cb + b * CPB
            ccb = c * (E_PAD // CH) + cb
            pltpu.async_copy(colX_h.at[pl.ds(ccb, CPB)], colb.at[p4],
                             semL[p4])
            pltpu.async_copy(rowX_h.at[pl.ds(cb, CPB)], rowb.at[p8],
                             semL[p4])
            pltpu.async_copy(valX_h.at[pl.ds(cb * (CH // 16), BLK // 16)],
                             valb.at[p4], semL[p4])

        def fire_gathers(src_h, p4):
            for j in range(CPB):
                pltpu.async_copy(src_h.at[colb.at[p4, j]],
                                 rows.at[p4, pl.ds(j * CH, CH)], semG[p4])

        def wait_gathers(src_h, p4):
            for j in range(CPB):
                pltpu.make_async_copy(
                    src_h.at[colb.at[p4, j]],
                    rows.at[p4, pl.ds(j * CH, CH)], semG[p4]).wait()

        def mult(p4):
            def mul_body(g, carry):
                v16 = valb[p4, g, :]
                for l in range(16):
                    b16 = _bcast_lane(v16, l)
                    e = g * 16 + l
                    rows[p4, e, :] = rows[p4, e, :] * b16
                return carry
            lax.fori_loop(0, BLK // 16, mul_body, 0)

        def fire_scat(p4, p8):
            for j in range(CPB):
                pltpu.async_copy(rows.at[p4, pl.ds(j * CH, CH)],
                                 acc.at[rowb.at[p8, j]], semS[p4], add=True)

        def wait_scat(p4, p8):
            for j in range(CPB):
                pltpu.make_async_copy(rows.at[p4, pl.ds(j * CH, CH)],
                                      acc.at[rowb.at[p8, j]], semS[p4]).wait()

        def wait_loads(b, p4, p8):
            cb = base_cb + b * CPB
            ccb = c * (E_PAD // CH) + cb
            pltpu.make_async_copy(colX_h.at[pl.ds(ccb, CPB)], colb.at[p4],
                                  semL[p4]).wait()
            pltpu.make_async_copy(rowX_h.at[pl.ds(cb, CPB)], rowb.at[p8],
                                  semL[p4]).wait()
            pltpu.make_async_copy(valX_h.at[pl.ds(cb * (CH // 16), BLK // 16)],
                                  valb.at[p4], semL[p4]).wait()

        def do_layer(src_h, dst_h, is_last):
            pltpu.sync_copy(zeros_h, acc.at[pl.ds(tile_lo, RPT)])
            plsc.subcore_barrier()

            for b0 in range(3):
                fire_loads(jnp.int32(b0), b0 % 4, b0 % 8)
            for b0 in range(2):
                wait_loads(jnp.int32(b0), b0 % 4, b0 % 8)
                fire_gathers(src_h, b0 % 4)

            def octo_body(t, carry):
                for u in range(8):
                    p4 = u % 4
                    p8 = u % 8
                    b = 8 * t + u
                    wait_gathers(src_h, p4)
                    mult(p4)
                    fire_scat(p4, p8)

                    @pl.when(b >= 2)
                    def _():
                        wait_scat((u + 2) % 4, (u + 2) % 8)

                    @pl.when(b + 3 < NB)
                    def _():
                        fire_loads(b + 3, (u + 3) % 4, (u + 3) % 8)

                    @pl.when(b + 2 < NB)
                    def _():
                        wait_loads(b + 2, (u + 2) % 4, (u + 2) % 8)
                        fire_gathers(src_h, (u + 2) % 4)
                return carry

            lax.fori_loop(0, NB // 8, octo_body, 0)
            wait_scat(2, 6)
            wait_scat(3, 7)
            plsc.subcore_barrier()

            if not is_last:
                pltpu.sync_copy(acc.at[pl.ds(tile_lo, RPT)],
                                dst_h.at[pl.ds(c * N_PAD + tile_lo, RPT)])
            else:
                def ep_body(z, carry):
                    bl = tile_lo + z * EZ
                    bg = c * N_PAD + bl
                    pltpu.sync_copy(acc.at[pl.ds(bl, EZ)], vA)
                    for other in (emb2_h, l1_h, l2_h):
                        pltpu.sync_copy(other.at[pl.ds(bg, EZ)], vB)

                        def add8(i, carry2):
                            for k in range(8):
                                e = i * 8 + k
                                vA[e, :] = vA[e, :] + vB[e, :]
                            return carry2
                        lax.fori_loop(0, EZ // 8, add8, 0)

                    def scl8(i, carry2):
                        for k in range(8):
                            e = i * 8 + k
                            vA[e, :] = vA[e, :] * 0.25
                        return carry2
                    lax.fori_loop(0, EZ // 8, scl8, 0)
                    pltpu.sync_copy(vA, sum_h.at[pl.ds(bg, EZ)])
                    return carry
                lax.fori_loop(0, NEZ, ep_body, 0)
            plsc.subcore_barrier()

        do_layer(emb2_h, l1_h, False)
        do_layer(l1_h, l2_h, False)
        do_layer(l2_h, None, True)

    return body(emb2, colX, rowX, valX, zeros)


def kernel(user_emb, item_emb, edge_index, edge_values):
    emb = jnp.concatenate([user_emb, item_emb], axis=0)
    padrows = jnp.zeros((N_PAD - N, H), jnp.float32)
    emb2 = jnp.concatenate(
        [emb[:, :H], padrows, emb[:, H:], padrows], axis=0)
    row = edge_index[0]
    col = edge_index[1]
    pad = E_PAD - E_RAW
    row_p = jnp.concatenate([row, jnp.zeros((pad,), jnp.int32)])
    col_p = jnp.concatenate([col, jnp.zeros((pad,), jnp.int32)])
    val_p = jnp.concatenate([edge_values, jnp.zeros((pad,), jnp.float32)])
    colX = jnp.concatenate([col_p, col_p + N_PAD]).reshape(2 * E_PAD // CH, CH)
    rowX = row_p.reshape(E_PAD // CH, CH)
    valX = val_p.reshape(E_PAD // 16, 16)
    zeros = jnp.zeros((RPT, H), jnp.float32)
    sum2, _l1, _l2 = _sc_propagate(emb2, colX, rowX, valX, zeros)
    final = jnp.stack([sum2[:N], sum2[N_PAD:N_PAD + N]], axis=1).reshape(N, D)
    return final[:N_U], final[N_U:]


# speedup vs baseline: 13.7629x; 1.0071x over previous
import functools

import jax
import jax.numpy as jnp
from jax import lax
from jax.experimental import pallas as pl
from jax.experimental.pallas import tpu as pltpu
from jax.experimental.pallas import tpu_sc as plsc

_BCAST_DNUMS = lax.GatherDimensionNumbers(
    offset_dims=(), collapsed_slice_dims=(0,), start_index_map=(0,))


def _bcast_lane(v16, lane):
    idx = jnp.full((16, 1), lane, dtype=jnp.int32)
    return lax.gather(v16, idx, _BCAST_DNUMS, (1,),
                      mode=lax.GatherScatterMode.PROMISE_IN_BOUNDS)


N_U = 50000
N_I = 50000
N = N_U + N_I
N_PAD = 100096
D = 32
H = 16
E_RAW = 1600000
CH = 128
CPB = 2
BLK = CH * CPB
N_TILES = 16
CPT = 800
NB = CPT // CPB
E_PAD = N_TILES * CPT * CH
RPT = N_PAD // N_TILES
EZ = 136
NEZ = RPT // EZ
GB = CPB * CH * H * 4
LB = 3 * CPB * CH * 4


def _sc_propagate(emb2, colX, rowX, valX, zeros):
    mesh = plsc.VectorSubcoreMesh(core_axis_name="c", subcore_axis_name="s")
    f32 = jnp.float32
    out_types = (
        jax.ShapeDtypeStruct((2 * N_PAD, H), f32),
        jax.ShapeDtypeStruct((2 * N_PAD, H), f32),
        jax.ShapeDtypeStruct((2 * N_PAD, H), f32),
    )
    scratch = (
        [pltpu.VMEM_SHARED((N_PAD, H), f32)]
        + [pltpu.VMEM((4, CPB, CH), jnp.int32)]
        + [pltpu.VMEM((8, CPB, CH), jnp.int32)]
        + [pltpu.VMEM((4, BLK // 16, 16), f32)]
        + [pltpu.VMEM((4, BLK, H), f32)]
        + [pltpu.VMEM((EZ, H), f32)] * 2
        + [pltpu.SemaphoreType.DMA] * 12
    )

    @functools.partial(
        pl.kernel, out_type=out_types, mesh=mesh, scratch_types=scratch,
        compiler_params=pltpu.CompilerParams(use_tc_tiling_on_sc=False))
    def body(emb2_h, colX_h, rowX_h, valX_h, zeros_h, sum_h, l1_h, l2_h,
             acc, colb, rowb, valb, rows, vA, vB,
             sg0, sg1, sg2, sg3, ss0, ss1, ss2, ss3, sl0, sl1, sl2, sl3):
        semG = [sg0, sg1, sg2, sg3]
        semS = [ss0, ss1, ss2, ss3]
        semL = [sl0, sl1, sl2, sl3]
        c = lax.axis_index("c")
        s = lax.axis_index("s")
        tile_lo = s * RPT
        base_cb = s * CPT

        def fire_loads(b, p4, p8):
            cb = base_cb + b * CPB
            ccb = c * (E_PAD // CH) + cb
            pltpu.async_copy(colX_h.at[pl.ds(ccb, CPB)], colb.at[p4],
                             semL[p4])
            pltpu.async_copy(rowX_h.at[pl.ds(cb, CPB)], rowb.at[p8],
                             semL[p4])
            pltpu.async_copy(valX_h.at[pl.ds(cb * (CH // 16), BLK // 16)],
                             valb.at[p4], semL[p4])

        def fire_gathers(src_h, p4):
            for j in range(CPB):
                pltpu.async_copy(src_h.at[colb.at[p4, j]],
                                 rows.at[p4, pl.ds(j * CH, CH)], semG[p4])

        def wait_gathers(src_h, p4):
            for j in range(CPB):
                pltpu.make_async_copy(
                    src_h.at[colb.at[p4, j]],
                    rows.at[p4, pl.ds(j * CH, CH)], semG[p4]).wait()

        def mult(p4):
            @plsc.parallel_loop(0, BLK // 16, 1, unroll=2)
            def _mul_body(g):
                v16 = valb[p4, g, :]
                for l in range(16):
                    b16 = _bcast_lane(v16, l)
                    e = g * 16 + l
                    rows[p4, e, :] = rows[p4, e, :] * b16

        def fire_scat(p4, p8):
            for j in range(CPB):
                pltpu.async_copy(rows.at[p4, pl.ds(j * CH, CH)],
                                 acc.at[rowb.at[p8, j]], semS[p4], add=True)

        def wait_scat(p4, p8):
            for j in range(CPB):
                pltpu.make_async_copy(rows.at[p4, pl.ds(j * CH, CH)],
                                      acc.at[rowb.at[p8, j]], semS[p4]).wait()

        def wait_loads(b, p4, p8):
            cb = base_cb + b * CPB
            ccb = c * (E_PAD // CH) + cb
            pltpu.make_async_copy(colX_h.at[pl.ds(ccb, CPB)], colb.at[p4],
                                  semL[p4]).wait()
            pltpu.make_async_copy(rowX_h.at[pl.ds(cb, CPB)], rowb.at[p8],
                                  semL[p4]).wait()
            pltpu.make_async_copy(valX_h.at[pl.ds(cb * (CH // 16), BLK // 16)],
                                  valb.at[p4], semL[p4]).wait()

        def do_layer(src_h, dst_h, is_last):
            pltpu.sync_copy(zeros_h, acc.at[pl.ds(tile_lo, RPT)])
            plsc.subcore_barrier()

            for b0 in range(3):
                fire_loads(jnp.int32(b0), b0 % 4, b0 % 8)
            for b0 in range(2):
                wait_loads(jnp.int32(b0), b0 % 4, b0 % 8)
                fire_gathers(src_h, b0 % 4)

            def octo_body(t, carry):
                for u in range(8):
                    p4 = u % 4
                    p8 = u % 8
                    b = 8 * t + u
                    wait_gathers(src_h, p4)
                    mult(p4)
                    fire_scat(p4, p8)

                    @pl.when(b >= 2)
                    def _():
                        wait_scat((u + 2) % 4, (u + 2) % 8)

                    @pl.when(b + 3 < NB)
                    def _():
                        fire_loads(b + 3, (u + 3) % 4, (u + 3) % 8)

                    @pl.when(b + 2 < NB)
                    def _():
                        wait_loads(b + 2, (u + 2) % 4, (u + 2) % 8)
                        fire_gathers(src_h, (u + 2) % 4)
                return carry

            lax.fori_loop(0, NB // 8, octo_body, 0)
            wait_scat(2, 6)
            wait_scat(3, 7)
            plsc.subcore_barrier()

            if not is_last:
                pltpu.sync_copy(acc.at[pl.ds(tile_lo, RPT)],
                                dst_h.at[pl.ds(c * N_PAD + tile_lo, RPT)])
            else:
                def ep_body(z, carry):
                    bl = tile_lo + z * EZ
                    bg = c * N_PAD + bl
                    pltpu.sync_copy(acc.at[pl.ds(bl, EZ)], vA)
                    for other in (emb2_h, l1_h, l2_h):
                        pltpu.sync_copy(other.at[pl.ds(bg, EZ)], vB)

                        def add8(i, carry2):
                            for k in range(8):
                                e = i * 8 + k
                                vA[e, :] = vA[e, :] + vB[e, :]
                            return carry2
                        lax.fori_loop(0, EZ // 8, add8, 0)

                    def scl8(i, carry2):
                        for k in range(8):
                            e = i * 8 + k
                            vA[e, :] = vA[e, :] * 0.25
                        return carry2
                    lax.fori_loop(0, EZ // 8, scl8, 0)
                    pltpu.sync_copy(vA, sum_h.at[pl.ds(bg, EZ)])
                    return carry
                lax.fori_loop(0, NEZ, ep_body, 0)
            plsc.subcore_barrier()

        do_layer(emb2_h, l1_h, False)
        do_layer(l1_h, l2_h, False)
        do_layer(l2_h, None, True)

    return body(emb2, colX, rowX, valX, zeros)


def kernel(user_emb, item_emb, edge_index, edge_values):
    emb = jnp.concatenate([user_emb, item_emb], axis=0)
    padrows = jnp.zeros((N_PAD - N, H), jnp.float32)
    emb2 = jnp.concatenate(
        [emb[:, :H], padrows, emb[:, H:], padrows], axis=0)
    row = edge_index[0]
    col = edge_index[1]
    pad = E_PAD - E_RAW
    row_p = jnp.concatenate([row, jnp.zeros((pad,), jnp.int32)])
    col_p = jnp.concatenate([col, jnp.zeros((pad,), jnp.int32)])
    val_p = jnp.concatenate([edge_values, jnp.zeros((pad,), jnp.float32)])
    colX = jnp.concatenate([col_p, col_p + N_PAD]).reshape(2 * E_PAD // CH, CH)
    rowX = row_p.reshape(E_PAD // CH, CH)
    valX = val_p.reshape(E_PAD // 16, 16)
    zeros = jnp.zeros((RPT, H), jnp.float32)
    sum2, _l1, _l2 = _sc_propagate(emb2, colX, rowX, valX, zeros)
    final = jnp.stack([sum2[:N], sum2[N_PAD:N_PAD + N]], axis=1).reshape(N, D)
    return final[:N_U], final[N_U:]
